# Initial kernel scaffold; baseline (speedup 1.0000x reference)
import jax
import jax.numpy as jnp
from jax.experimental import pallas as pl


def kernel(X, table):
    raise NotImplementedError("write your pallas kernel here")


# speedup vs baseline: 1.2612x; 1.2612x over previous
import functools

import jax
import jax.numpy as jnp
from jax import lax
from jax.experimental import pallas as pl
from jax.experimental.pallas import tpu as pltpu
from jax.experimental.pallas import tpu_sc as plsc

D = 1000
DPAD = 1024
B = 1024 * 50
NC, NS = 2, 16
NW = NC * NS
B_PER_W = B // NW
CHUNK = 80
N_CHUNKS = B_PER_W // CHUNK


@functools.partial(jax.jit, static_argnums=())
def _sc_gather(table, idx):
    mesh = plsc.VectorSubcoreMesh(core_axis_name="c", subcore_axis_name="s")

    @functools.partial(
        pl.kernel,
        mesh=mesh,
        out_type=jax.ShapeDtypeStruct((B, D), jnp.float32),
        scratch_types=[
            pltpu.VMEM((B_PER_W,), jnp.int32),
            pltpu.VMEM((CHUNK, DPAD), jnp.float32),
            pltpu.VMEM((CHUNK, D - 896), jnp.float32),
            pltpu.SemaphoreType.DMA,
        ],
    )
    def k(table_hbm, idx_hbm, out_hbm, idx_v, rows_v, tail_v, sem):
        wid = lax.axis_index("s") * NC + lax.axis_index("c")
        base = wid * B_PER_W
        pltpu.sync_copy(idx_hbm.at[pl.ds(base, B_PER_W)], idx_v)

        def body(c, carry):
            off = c * CHUNK
            pltpu.async_copy(
                table_hbm.at[idx_v.at[pl.ds(off, CHUNK)]], rows_v, sem
            ).wait()
            for b in range(7):
                pltpu.sync_copy(
                    rows_v.at[:, pl.ds(b * 128, 128)],
                    out_hbm.at[pl.ds(base + off, CHUNK), pl.ds(b * 128, 128)],
                )

            def tail_row(i, carry2):
                for t in range(6):
                    tail_v[i, pl.ds(t * 16, 16)] = rows_v[i, pl.ds(896 + t * 16, 16)]
                tail_v[i, pl.ds(88, 16)] = rows_v[i, pl.ds(984, 16)]
                return carry2

            lax.fori_loop(0, CHUNK, tail_row, 0)
            pltpu.sync_copy(
                tail_v, out_hbm.at[pl.ds(base + off, CHUNK), pl.ds(896, 104)]
            )
            return carry

        lax.fori_loop(0, N_CHUNKS, body, 0)

    return k(table, idx)


def kernel(X, table):
    idx = X.reshape(-1)
    table_pad = jnp.pad(table, ((0, 0), (0, DPAD - D)))
    return _sc_gather(table_pad, idx)


# speedup vs baseline: 1.3387x; 1.0614x over previous
import functools

import jax
import jax.numpy as jnp
from jax import lax
from jax.experimental import pallas as pl
from jax.experimental.pallas import tpu as pltpu
from jax.experimental.pallas import tpu_sc as plsc

V = 1000
D = 1000
DPAD = 1024
BULK = 896
TAIL = D - BULK
B = 1024 * 50
NC, NS = 2, 16
NW = NC * NS
B_PER_W = B // NW
CHUNK = 40
N_GROUPS = B_PER_W // (2 * CHUNK)


def _sc_gather(table, idx):
    mesh = plsc.VectorSubcoreMesh(core_axis_name="c", subcore_axis_name="s")

    @functools.partial(
        pl.kernel,
        mesh=mesh,
        out_type=jax.ShapeDtypeStruct((B, D), jnp.float32),
        scratch_types=[
            pltpu.VMEM((B_PER_W,), jnp.int32),
            pltpu.VMEM((CHUNK, DPAD), jnp.float32),
            pltpu.VMEM((CHUNK, DPAD), jnp.float32),
            pltpu.VMEM((CHUNK, TAIL), jnp.float32),
            pltpu.VMEM((CHUNK, TAIL), jnp.float32),
            pltpu.SemaphoreType.DMA,
            pltpu.SemaphoreType.DMA,
            pltpu.SemaphoreType.DMA,
            pltpu.SemaphoreType.DMA,
        ],
    )
    def k(table_hbm, idx_hbm, out_hbm, idx_v,
          rows0, rows1, tail0, tail1, sem_g0, sem_g1, sem_w0, sem_w1):
        cid = lax.axis_index("c")
        sid = lax.axis_index("s")
        wid = sid * NC + cid
        base = wid * B_PER_W
        pltpu.sync_copy(idx_hbm.at[pl.ds(base, B_PER_W)], idx_v)

        rows = (rows0, rows1)
        tails = (tail0, tail1)
        sem_g = (sem_g0, sem_g1)
        sem_w = (sem_w0, sem_w1)

        def gather_start(c, p):
            pltpu.async_copy(
                table_hbm.at[idx_v.at[pl.ds(c * CHUNK, CHUNK)]], rows[p], sem_g[p]
            )

        def gather_wait(c, p):
            pltpu.make_async_copy(
                table_hbm.at[idx_v.at[pl.ds(c * CHUNK, CHUNK)]], rows[p], sem_g[p]
            ).wait()

        def repack_tail(p):
            def row(i, carry):
                for t in range(6):
                    tails[p][i, pl.ds(t * 16, 16)] = rows[p][i, pl.ds(BULK + t * 16, 16)]
                tails[p][i, pl.ds(TAIL - 16, 16)] = rows[p][i, pl.ds(D - 16, 16)]
                return carry

            lax.fori_loop(0, CHUNK, row, 0)

        def write_start(c, p):
            o = base + c * CHUNK
            pltpu.async_copy(
                rows[p].at[:, pl.ds(0, BULK)],
                out_hbm.at[pl.ds(o, CHUNK), pl.ds(0, BULK)],
                sem_w[p],
            )
            pltpu.async_copy(
                tails[p], out_hbm.at[pl.ds(o, CHUNK), pl.ds(BULK, TAIL)], sem_w[p]
            )

        def write_wait(c, p):
            o = base + c * CHUNK
            pltpu.make_async_copy(
                rows[p].at[:, pl.ds(0, BULK)],
                out_hbm.at[pl.ds(o, CHUNK), pl.ds(0, BULK)],
                sem_w[p],
            ).wait()
            pltpu.make_async_copy(
                tails[p], out_hbm.at[pl.ds(o, CHUNK), pl.ds(BULK, TAIL)], sem_w[p]
            ).wait()

        gather_start(0, 0)

        def group(g, carry):
            c0 = 2 * g
            c1 = c0 + 1
            gather_wait(c0, 0)
            repack_tail(0)
            write_start(c0, 0)

            @pl.when(g > 0)
            def _():
                write_wait(c0 - 1, 1)

            gather_start(c1, 1)
            gather_wait(c1, 1)
            repack_tail(1)
            write_start(c1, 1)
            write_wait(c0, 0)

            @pl.when(g < N_GROUPS - 1)
            def _():
                gather_start(c0 + 2, 0)

            return carry

        lax.fori_loop(0, N_GROUPS, group, 0)
        write_wait(2 * N_GROUPS - 1, 1)

    return k(table, idx)


def kernel(X, table):
    idx = X.reshape(-1)
    table_pad = jnp.pad(table, ((0, 0), (0, DPAD - D)))
    return _sc_gather(table_pad, idx)
